# baseline (speedup 1.0000x reference)
import functools

import jax
import jax.numpy as jnp
from jax import lax
from jax.experimental import pallas as pl
from jax.experimental.pallas import tpu as pltpu
from jax.experimental.pallas import tpu_sc as plsc

N = 16384
B = 64
NNZ_PER_ROW = 16
CH = 16
CHN = CH * NNZ_PER_ROW
GRP = 4

_GATHER_DIM_NUMS = lax.GatherDimensionNumbers(
    offset_dims=(), collapsed_slice_dims=(0,), start_index_map=(0,))


def _splat_lane(vec, j):
    idx = jnp.full((16, 1), j, dtype=jnp.int32)
    return lax.gather(vec, idx, _GATHER_DIM_NUMS, slice_sizes=(1,),
                      mode=lax.GatherScatterMode.PROMISE_IN_BOUNDS)


def _make_kernel():
    info = plsc.get_sparse_core_info()
    nc, ns = info.num_cores, info.num_subcores
    nw = nc * ns
    rows_per_w = N // nw
    n_chunks = rows_per_w // CH
    nnz_per_w = rows_per_w * NNZ_PER_ROW

    mesh = plsc.VectorSubcoreMesh(core_axis_name="c", subcore_axis_name="s")

    @functools.partial(
        pl.kernel,
        out_type=jax.ShapeDtypeStruct((B, N), jnp.float32),
        mesh=mesh,
        compiler_params=pltpu.CompilerParams(use_tc_tiling_on_sc=False,
                                             needs_layout_passes=False),
        scratch_types=[
            pltpu.VMEM((nnz_per_w,), jnp.int32),
            pltpu.VMEM((nnz_per_w,), jnp.float32),
            pltpu.VMEM((rows_per_w,), jnp.float32),
            [pltpu.VMEM((CHN, B), jnp.bfloat16)] * 4,
            [pltpu.VMEM((B, CH), jnp.float32)] * 4,
            [pltpu.SemaphoreType.DMA] * 4,
            [pltpu.SemaphoreType.DMA] * 4,
        ],
    )
    def k(xt_hbm, col_hbm, val_hbm, bias_hbm, out_hbm,
          col_v, val_v, bias_v, gbufs, obufs, gsems, ssems):
        wid = lax.axis_index("s") * nc + lax.axis_index("c")
        row0 = wid * rows_per_w

        two_iota = 2 * lax.iota(jnp.int32, 16)

        pltpu.sync_copy(col_hbm.at[pl.ds(wid * nnz_per_w, nnz_per_w)], col_v)
        pltpu.sync_copy(val_hbm.at[pl.ds(wid * nnz_per_w, nnz_per_w)], val_v)
        pltpu.sync_copy(bias_hbm.at[pl.ds(row0, rows_per_w)], bias_v)

        def fire_gather(t, gbuf, sem):
            pltpu.async_copy(xt_hbm.at[col_v.at[pl.ds(t * CHN, 128)]],
                             gbuf.at[pl.ds(0, 128)], sem)
            pltpu.async_copy(xt_hbm.at[col_v.at[pl.ds(t * CHN + 128, 128)]],
                             gbuf.at[pl.ds(128, 128)], sem)

        def wait_gather(gbuf, sem):
            pltpu.make_async_copy(xt_hbm.at[pl.ds(0, 128)],
                                  gbuf.at[pl.ds(0, 128)], sem).wait()
            pltpu.make_async_copy(xt_hbm.at[pl.ds(0, 128)],
                                  gbuf.at[pl.ds(128, 128)], sem).wait()

        def wait_store(obuf, sem):
            pltpu.make_async_copy(obuf, out_hbm.at[:, pl.ds(0, CH)],
                                  sem).wait()

        col_idx = [two_iota, two_iota + 1, two_iota + 32, two_iota + 33]

        def compute(t, gbuf, obuf, sem):
            bv = bias_v[pl.ds(t * CH, CH)]

            @plsc.parallel_loop(0, CH, unroll=8)
            def _row(i):
                vv = val_v[pl.ds((t * CH + i) * NNZ_PER_ROW, 16)]
                bb = _splat_lane(bv, i)
                accs = [bb, bb, bb, bb]
                for g0 in range(0, NNZ_PER_ROW, GRP):
                    s_lo = s_hi = None
                    for j in range(g0, g0 + GRP):
                        wf = _splat_lane(vv, j)
                        wv = plsc.pack(wf, wf,
                                       format=plsc.PackFormat.INTERLEAVED)
                        gr = i * NNZ_PER_ROW + j
                        p_lo = wv * gbuf[gr, pl.ds(0, 32)]
                        p_hi = wv * gbuf[gr, pl.ds(32, 32)]
                        s_lo = p_lo if s_lo is None else s_lo + p_lo
                        s_hi = p_hi if s_hi is None else s_hi + p_hi
                    lo = plsc.unpack(s_lo, format=plsc.PackFormat.INTERLEAVED)
                    hi = plsc.unpack(s_hi, format=plsc.PackFormat.INTERLEAVED)
                    for c, part in enumerate((lo[0], lo[1], hi[0], hi[1])):
                        accs[c] = accs[c] + part
                row_idx = jnp.full((16,), i, jnp.int32)
                for c in range(4):
                    plsc.store_scatter(obuf, [col_idx[c], row_idx], accs[c])

            pltpu.async_copy(obuf, out_hbm.at[:, pl.ds(row0 + t * CH, CH)],
                             sem)

        for u in range(3):
            fire_gather(u, gbufs[u], gsems[u])

        def body(qq, _):
            for u in range(4):
                t = 4 * qq + u
                wait_gather(gbufs[u], gsems[u])

                @pl.when(qq > 0)
                def _():
                    wait_store(obufs[u], ssems[u])

                @pl.when(t + 3 < n_chunks)
                def _():
                    fire_gather(t + 3, gbufs[(u + 3) % 4],
                                gsems[(u + 3) % 4])

                compute(t, gbufs[u], obufs[u], ssems[u])
            return ()

        lax.fori_loop(0, n_chunks // 4, body, ())
        for u in range(4):
            wait_store(obufs[u], ssems[u])

    return k


def kernel(X, values, bias, crow_indices, col_indices):
    del crow_indices
    xtb = X.T.astype(jnp.bfloat16).reshape(N, B)
    return _make_kernel()(xtb, col_indices, values, bias)


# speedup vs baseline: 1.0302x; 1.0302x over previous
import functools

import jax
import jax.numpy as jnp
from jax import lax
from jax.experimental import pallas as pl
from jax.experimental.pallas import tpu as pltpu
from jax.experimental.pallas import tpu_sc as plsc

N = 16384
B = 64
NNZ_PER_ROW = 16
CH = 32
CHN = CH * NNZ_PER_ROW
GRP = 4

_GATHER_DIM_NUMS = lax.GatherDimensionNumbers(
    offset_dims=(), collapsed_slice_dims=(0,), start_index_map=(0,))


def _splat_lane(vec, j):
    idx = jnp.full((16, 1), j, dtype=jnp.int32)
    return lax.gather(vec, idx, _GATHER_DIM_NUMS, slice_sizes=(1,),
                      mode=lax.GatherScatterMode.PROMISE_IN_BOUNDS)


def _make_kernel():
    info = plsc.get_sparse_core_info()
    nc, ns = info.num_cores, info.num_subcores
    nw = nc * ns
    rows_per_w = N // nw
    n_chunks = rows_per_w // CH
    nnz_per_w = rows_per_w * NNZ_PER_ROW

    mesh = plsc.VectorSubcoreMesh(core_axis_name="c", subcore_axis_name="s")

    @functools.partial(
        pl.kernel,
        out_type=jax.ShapeDtypeStruct((B, N), jnp.float32),
        mesh=mesh,
        compiler_params=pltpu.CompilerParams(use_tc_tiling_on_sc=False,
                                             needs_layout_passes=False),
        scratch_types=[
            pltpu.VMEM((nnz_per_w,), jnp.int32),
            pltpu.VMEM((nnz_per_w,), jnp.float32),
            pltpu.VMEM((rows_per_w,), jnp.float32),
            [pltpu.VMEM((CHN, B), jnp.bfloat16)] * 2,
            [pltpu.VMEM((B, CH), jnp.float32)] * 2,
            [pltpu.SemaphoreType.DMA] * 2,
            [pltpu.SemaphoreType.DMA] * 2,
        ],
    )
    def k(xt_hbm, col_hbm, val_hbm, bias_hbm, out_hbm,
          col_v, val_v, bias_v, gbufs, obufs, gsems, ssems):
        wid = lax.axis_index("s") * nc + lax.axis_index("c")
        row0 = wid * rows_per_w

        two_iota = 2 * lax.iota(jnp.int32, 16)

        pltpu.sync_copy(col_hbm.at[pl.ds(wid * nnz_per_w, nnz_per_w)], col_v)
        pltpu.sync_copy(val_hbm.at[pl.ds(wid * nnz_per_w, nnz_per_w)], val_v)
        pltpu.sync_copy(bias_hbm.at[pl.ds(row0, rows_per_w)], bias_v)

        def fire_gather(t, gbuf, sem):
            for h in range(CHN // 128):
                pltpu.async_copy(
                    xt_hbm.at[col_v.at[pl.ds(t * CHN + h * 128, 128)]],
                    gbuf.at[pl.ds(h * 128, 128)], sem)

        def wait_gather(gbuf, sem):
            for h in range(CHN // 128):
                pltpu.make_async_copy(xt_hbm.at[pl.ds(0, 128)],
                                      gbuf.at[pl.ds(h * 128, 128)],
                                      sem).wait()

        def wait_store(obuf, sem):
            pltpu.make_async_copy(obuf, out_hbm.at[:, pl.ds(0, CH)],
                                  sem).wait()

        col_idx = [two_iota, two_iota + 1, two_iota + 32, two_iota + 33]

        def compute(t, gbuf, obuf, sem):
            @plsc.parallel_loop(0, CH, unroll=8)
            def _row(i):
                vv = val_v[pl.ds((t * CH + i) * NNZ_PER_ROW, 16)]
                bv = bias_v[pl.ds(t * CH + (i // 16) * 16, 16)]
                bb = _splat_lane(bv, i % 16)
                accs = [bb, bb, bb, bb]
                for g0 in range(0, NNZ_PER_ROW, GRP):
                    s_lo = s_hi = None
                    for j in range(g0, g0 + GRP):
                        wf = _splat_lane(vv, j)
                        wv = plsc.pack(wf, wf,
                                       format=plsc.PackFormat.INTERLEAVED)
                        gr = i * NNZ_PER_ROW + j
                        p_lo = wv * gbuf[gr, pl.ds(0, 32)]
                        p_hi = wv * gbuf[gr, pl.ds(32, 32)]
                        s_lo = p_lo if s_lo is None else s_lo + p_lo
                        s_hi = p_hi if s_hi is None else s_hi + p_hi
                    lo = plsc.unpack(s_lo, format=plsc.PackFormat.INTERLEAVED)
                    hi = plsc.unpack(s_hi, format=plsc.PackFormat.INTERLEAVED)
                    for c, part in enumerate((lo[0], lo[1], hi[0], hi[1])):
                        accs[c] = accs[c] + part
                row_idx = jnp.full((16,), i, jnp.int32)
                for c in range(4):
                    plsc.store_scatter(obuf, [col_idx[c], row_idx], accs[c])

            pltpu.async_copy(obuf, out_hbm.at[:, pl.ds(row0 + t * CH, CH)],
                             sem)

        fire_gather(0, gbufs[0], gsems[0])

        def body(tt, _):
            for u in range(2):
                t = 2 * tt + u
                wait_gather(gbufs[u], gsems[u])

                @pl.when(t >= 2)
                def _():
                    wait_store(obufs[u], ssems[u])

                @pl.when(t + 1 < n_chunks)
                def _():
                    fire_gather(t + 1, gbufs[(u + 1) % 2],
                                gsems[(u + 1) % 2])

                compute(t, gbufs[u], obufs[u], ssems[u])
            return ()

        lax.fori_loop(0, n_chunks // 2, body, ())
        for u in range(2):
            wait_store(obufs[u], ssems[u])

    return k


def kernel(X, values, bias, crow_indices, col_indices):
    del crow_indices
    xtb = X.T.astype(jnp.bfloat16).reshape(N, B)
    return _make_kernel()(xtb, col_indices, values, bias)
